# baseline (speedup 1.0000x reference)
import functools

import jax
import jax.numpy as jnp
from jax import lax
from jax.experimental import pallas as pl
from jax.experimental.pallas import tpu as pltpu
from jax.experimental.pallas import tpu_sc as plsc

NC = 2
NS = 16
NW = NC * NS
L = 16
U = 5


def _wid():
    return lax.axis_index("s") * NC + lax.axis_index("c")


def _pass1_body(M, T, B, BP, seg_hbm, qa_hbm, sums_hbm, cnts_hbm,
                seg0, seg1, qa0, qa1, accs_a, accs_b, accc_a, accc_b,
                row_v, sems):
    wid = _wid()
    base = wid * M
    lane_off = lax.iota(jnp.int32, L) * BP
    ones = jnp.ones((L,), jnp.float32)
    zeros = jnp.zeros((L,), jnp.float32)
    bufs = ((seg0, qa0), (seg1, qa1))
    nchunks = M // T

    def issue(k, slot):
        off = base + k * T
        sb, qb = bufs[slot]
        c1 = pltpu.async_copy(seg_hbm.at[pl.ds(off, T)], sb, sems.at[slot])
        c2 = pltpu.async_copy(qa_hbm.at[pl.ds(off, T)], qb, sems.at[slot])
        return (c1, c2)

    copies = [issue(0, 0), None]

    def zero_body(j, _):
        for u in range(U):
            sl = pl.ds((j * U + u) * L, L)
            accs_a[sl] = zeros
            accs_b[sl] = zeros
            accc_a[sl] = zeros
            accc_b[sl] = zeros
        return 0

    lax.fori_loop(0, (L * BP) // (L * U), zero_body, 0)

    for k in range(nchunks):
        slot = k % 2
        if k + 1 < nchunks:
            copies[(k + 1) % 2] = issue(k + 1, (k + 1) % 2)
        for c in copies[slot]:
            c.wait()
        sb, qb = bufs[slot]

        def vec_body(v, _):
            for u in range(U):
                sl = pl.ds((v * U + u) * L, L)
                s = sb[sl]
                q = qb[sl]
                idx = lane_off + s
                plsc.addupdate_scatter(accs_a if u % 2 == 0 else accs_b,
                                       [idx], q)
                plsc.addupdate_scatter(accc_a if u % 2 == 0 else accc_b,
                                       [idx], ones)
            return 0

        lax.fori_loop(0, T // (L * U), vec_body, 0)

    def red_sums(j, _):
        sl = pl.ds(j * L, L)
        tot = accs_a[sl] + accs_b[sl]
        for i in range(1, L):
            tot = tot + accs_a[pl.ds(i * BP + j * L, L)]
            tot = tot + accs_b[pl.ds(i * BP + j * L, L)]
        row_v[sl] = tot
        return 0

    lax.fori_loop(0, B // L, red_sums, 0)
    pltpu.sync_copy(row_v, sums_hbm.at[pl.ds(wid * B, B)])

    def red_cnts(j, _):
        sl = pl.ds(j * L, L)
        tot = accc_a[sl] + accc_b[sl]
        for i in range(1, L):
            tot = tot + accc_a[pl.ds(i * BP + j * L, L)]
            tot = tot + accc_b[pl.ds(i * BP + j * L, L)]
        row_v[sl] = tot
        return 0

    lax.fori_loop(0, B // L, red_cnts, 0)
    pltpu.sync_copy(row_v, cnts_hbm.at[pl.ds(wid * B, B)])


def _pass2_body(M, T, B, BP, seg_hbm, qa_hbm, q_hbm, sums_hbm, cnts_hbm,
                out_hbm, rawq_hbm, seg0, seg1, qa0, qa1, out0, out1,
                big_v, corr_rep, qv_v, raw_v, sems, osems):
    wid = _wid()
    base = wid * M
    lane_off = lax.iota(jnp.int32, L) * BP
    bufs = ((seg0, qa0, out0), (seg1, qa1, out1))
    nchunks = M // T

    def issue(k, slot):
        off = base + k * T
        sb, qb, _ = bufs[slot]
        c1 = pltpu.async_copy(seg_hbm.at[pl.ds(off, T)], sb, sems.at[slot])
        c2 = pltpu.async_copy(qa_hbm.at[pl.ds(off, T)], qb, sems.at[slot])
        return (c1, c2)

    copies = [issue(0, 0), None]

    pltpu.sync_copy(q_hbm, qv_v)
    pltpu.sync_copy(sums_hbm, big_v)

    def comb_sums(j, _):
        sl = pl.ds(j * L, L)
        s = big_v[sl]
        for i in range(1, NW):
            s = s + big_v[pl.ds(i * B + j * L, L)]
        raw_v[sl] = s
        return 0

    lax.fori_loop(0, B // L, comb_sums, 0)
    pltpu.sync_copy(cnts_hbm, big_v)

    def comb_cnts(j, _):
        sl = pl.ds(j * L, L)
        c = big_v[sl]
        for i in range(1, NW):
            c = c + big_v[pl.ds(i * B + j * L, L)]
        corr = (qv_v[sl] - raw_v[sl]) / c
        for i in range(L):
            corr_rep[pl.ds(i * BP + j * L, L)] = corr
        return 0

    lax.fori_loop(0, B // L, comb_cnts, 0)

    @pl.when(wid == 0)
    def _():
        pltpu.sync_copy(raw_v, rawq_hbm)

    outcp = [None, None]
    for k in range(nchunks):
        slot = k % 2
        if k + 1 < nchunks:
            copies[(k + 1) % 2] = issue(k + 1, (k + 1) % 2)
        for c in copies[slot]:
            c.wait()
        if outcp[slot] is not None:
            outcp[slot].wait()
        sb, qb, ob = bufs[slot]

        def vec_body(v, _):
            for u in range(U):
                sl = pl.ds((v * U + u) * L, L)
                s = sb[sl]
                q = qb[sl]
                c = plsc.load_gather(corr_rep, [lane_off + s])
                ob[sl] = q + c
            return 0

        lax.fori_loop(0, T // (L * U), vec_body, 0)
        off = base + k * T
        outcp[slot] = pltpu.async_copy(ob, out_hbm.at[pl.ds(off, T)],
                                       osems.at[slot])
    for cp in outcp:
        if cp is not None:
            cp.wait()


def kernel(Za, Qa, Q, batch_seg):
    del Za
    N = Qa.shape[0]
    B = Q.shape[0]
    assert N % NW == 0
    M = N // NW
    T = 10000
    assert M % T == 0 and (T // L) % U == 0

    seg = batch_seg.astype(jnp.int32)
    qa = Qa.astype(jnp.float32)

    BP = B + 1

    mesh = plsc.VectorSubcoreMesh(core_axis_name="c", subcore_axis_name="s")

    pass1 = pl.kernel(
        functools.partial(_pass1_body, M, T, B, BP),
        out_type=(
            jax.ShapeDtypeStruct((NW * B,), jnp.float32),
            jax.ShapeDtypeStruct((NW * B,), jnp.float32),
        ),
        mesh=mesh,
        compiler_params=pltpu.CompilerParams(needs_layout_passes=False),
        scratch_types=[
            pltpu.VMEM((T,), jnp.int32),
            pltpu.VMEM((T,), jnp.int32),
            pltpu.VMEM((T,), jnp.float32),
            pltpu.VMEM((T,), jnp.float32),
            pltpu.VMEM((L * BP,), jnp.float32),
            pltpu.VMEM((L * BP,), jnp.float32),
            pltpu.VMEM((L * BP,), jnp.float32),
            pltpu.VMEM((L * BP,), jnp.float32),
            pltpu.VMEM((B,), jnp.float32),
            pltpu.SemaphoreType.DMA((2,)),
        ],
    )
    sums, cnts = pass1(seg, qa)

    pass2 = pl.kernel(
        functools.partial(_pass2_body, M, T, B, BP),
        out_type=(
            jax.ShapeDtypeStruct((N,), jnp.float32),
            jax.ShapeDtypeStruct((B,), jnp.float32),
        ),
        mesh=mesh,
        compiler_params=pltpu.CompilerParams(needs_layout_passes=False),
        scratch_types=[
            pltpu.VMEM((T,), jnp.int32),
            pltpu.VMEM((T,), jnp.int32),
            pltpu.VMEM((T,), jnp.float32),
            pltpu.VMEM((T,), jnp.float32),
            pltpu.VMEM((T,), jnp.float32),
            pltpu.VMEM((T,), jnp.float32),
            pltpu.VMEM((NW * B,), jnp.float32),
            pltpu.VMEM((L * BP,), jnp.float32),
            pltpu.VMEM((B,), jnp.float32),
            pltpu.VMEM((B,), jnp.float32),
            pltpu.SemaphoreType.DMA((2,)),
            pltpu.SemaphoreType.DMA((2,)),
        ],
    )
    out, rawq = pass2(seg, qa, Q.astype(jnp.float32), sums, cnts)
    return (out, rawq)


# speedup vs baseline: 1.2227x; 1.2227x over previous
import functools

import jax
import jax.numpy as jnp
from jax import lax
from jax.experimental import pallas as pl
from jax.experimental.pallas import tpu as pltpu
from jax.experimental.pallas import tpu_sc as plsc

NC = 2
NS = 16
NW = NC * NS
L = 16
U = 5


def _wid():
    return lax.axis_index("s") * NC + lax.axis_index("c")


def _pass1_body(M, T, B, BP, seg_hbm, qa_hbm, sums_hbm, cnts_hbm,
                seg0, seg1, qa0, qa1, accs_a, accs_b, accc_a, accc_b,
                row_v, sems):
    wid = _wid()
    base = wid * M
    lane_off = lax.iota(jnp.int32, L) * BP
    ones = jnp.ones((L,), jnp.float32)
    zeros = jnp.zeros((L,), jnp.float32)
    bufs = ((seg0, qa0), (seg1, qa1))
    nchunks = M // T

    def issue(k, slot):
        off = base + k * T
        sb, qb = bufs[slot]
        c1 = pltpu.async_copy(seg_hbm.at[pl.ds(off, T)], sb, sems.at[slot])
        c2 = pltpu.async_copy(qa_hbm.at[pl.ds(off, T)], qb, sems.at[slot])
        return (c1, c2)

    copies = [issue(0, 0), None]

    @plsc.parallel_loop(0, (L * BP) // L, unroll=U)
    def zero_body(j):
        sl = pl.ds(j * L, L)
        accs_a[sl] = zeros
        accs_b[sl] = zeros
        accc_a[sl] = zeros
        accc_b[sl] = zeros

    for k in range(nchunks):
        slot = k % 2
        if k + 1 < nchunks:
            copies[(k + 1) % 2] = issue(k + 1, (k + 1) % 2)
        for c in copies[slot]:
            c.wait()
        sb, qb = bufs[slot]

        def vec_body(v, _):
            for u in range(U):
                sl = pl.ds((v * U + u) * L, L)
                s = sb[sl]
                q = qb[sl]
                idx = lane_off + s
                plsc.addupdate_scatter(accs_a if u % 2 == 0 else accs_b,
                                       [idx], q)
                plsc.addupdate_scatter(accc_a if u % 2 == 0 else accc_b,
                                       [idx], ones)
            return 0

        lax.fori_loop(0, T // (L * U), vec_body, 0)

    @plsc.parallel_loop(0, B // L, unroll=2)
    def red_sums(j):
        sl = pl.ds(j * L, L)
        tot = accs_a[sl] + accs_b[sl]
        for i in range(1, L):
            tot = tot + accs_a[pl.ds(i * BP + j * L, L)]
            tot = tot + accs_b[pl.ds(i * BP + j * L, L)]
        row_v[sl] = tot

    pltpu.sync_copy(row_v, sums_hbm.at[pl.ds(wid * B, B)])

    @plsc.parallel_loop(0, B // L, unroll=2)
    def red_cnts(j):
        sl = pl.ds(j * L, L)
        tot = accc_a[sl] + accc_b[sl]
        for i in range(1, L):
            tot = tot + accc_a[pl.ds(i * BP + j * L, L)]
            tot = tot + accc_b[pl.ds(i * BP + j * L, L)]
        row_v[sl] = tot

    pltpu.sync_copy(row_v, cnts_hbm.at[pl.ds(wid * B, B)])


def _pass2_body(M, T, B, BP, seg_hbm, qa_hbm, q_hbm, sums_hbm, cnts_hbm,
                out_hbm, rawq_hbm, seg0, seg1, qa0, qa1, out0, out1,
                big_v, corr_rep, qv_v, raw_v, sems, osems):
    wid = _wid()
    base = wid * M
    lane_off = lax.iota(jnp.int32, L) * BP
    bufs = ((seg0, qa0, out0), (seg1, qa1, out1))
    nchunks = M // T

    def issue(k, slot):
        off = base + k * T
        sb, qb, _ = bufs[slot]
        c1 = pltpu.async_copy(seg_hbm.at[pl.ds(off, T)], sb, sems.at[slot])
        c2 = pltpu.async_copy(qa_hbm.at[pl.ds(off, T)], qb, sems.at[slot])
        return (c1, c2)

    copies = [issue(0, 0), None]

    pltpu.sync_copy(q_hbm, qv_v)
    pltpu.sync_copy(sums_hbm, big_v)

    @plsc.parallel_loop(0, B // L, unroll=2)
    def comb_sums(j):
        sl = pl.ds(j * L, L)
        s = big_v[sl]
        for i in range(1, NW):
            s = s + big_v[pl.ds(i * B + j * L, L)]
        raw_v[sl] = s

    pltpu.sync_copy(cnts_hbm, big_v)

    @plsc.parallel_loop(0, B // L, unroll=2)
    def comb_cnts(j):
        sl = pl.ds(j * L, L)
        c = big_v[sl]
        for i in range(1, NW):
            c = c + big_v[pl.ds(i * B + j * L, L)]
        corr = (qv_v[sl] - raw_v[sl]) / c
        for i in range(L):
            corr_rep[pl.ds(i * BP + j * L, L)] = corr

    @pl.when(wid == 0)
    def _():
        pltpu.sync_copy(raw_v, rawq_hbm)

    outcp = [None, None]
    for k in range(nchunks):
        slot = k % 2
        if k + 1 < nchunks:
            copies[(k + 1) % 2] = issue(k + 1, (k + 1) % 2)
        for c in copies[slot]:
            c.wait()
        if outcp[slot] is not None:
            outcp[slot].wait()
        sb, qb, ob = bufs[slot]

        @plsc.parallel_loop(0, T // L, unroll=U)
        def vec_body(v):
            sl = pl.ds(v * L, L)
            s = sb[sl]
            q = qb[sl]
            c = plsc.load_gather(corr_rep, [lane_off + s])
            ob[sl] = q + c

        off = base + k * T
        outcp[slot] = pltpu.async_copy(ob, out_hbm.at[pl.ds(off, T)],
                                       osems.at[slot])
    for cp in outcp:
        if cp is not None:
            cp.wait()


def kernel(Za, Qa, Q, batch_seg):
    del Za
    N = Qa.shape[0]
    B = Q.shape[0]
    assert N % NW == 0
    M = N // NW
    T = 10000
    assert M % T == 0 and (T // L) % U == 0

    seg = batch_seg.astype(jnp.int32)
    qa = Qa.astype(jnp.float32)

    BP = B + 1

    mesh = plsc.VectorSubcoreMesh(core_axis_name="c", subcore_axis_name="s")

    pass1 = pl.kernel(
        functools.partial(_pass1_body, M, T, B, BP),
        out_type=(
            jax.ShapeDtypeStruct((NW * B,), jnp.float32),
            jax.ShapeDtypeStruct((NW * B,), jnp.float32),
        ),
        mesh=mesh,
        compiler_params=pltpu.CompilerParams(needs_layout_passes=False),
        scratch_types=[
            pltpu.VMEM((T,), jnp.int32),
            pltpu.VMEM((T,), jnp.int32),
            pltpu.VMEM((T,), jnp.float32),
            pltpu.VMEM((T,), jnp.float32),
            pltpu.VMEM((L * BP,), jnp.float32),
            pltpu.VMEM((L * BP,), jnp.float32),
            pltpu.VMEM((L * BP,), jnp.float32),
            pltpu.VMEM((L * BP,), jnp.float32),
            pltpu.VMEM((B,), jnp.float32),
            pltpu.SemaphoreType.DMA((2,)),
        ],
    )
    sums, cnts = pass1(seg, qa)

    pass2 = pl.kernel(
        functools.partial(_pass2_body, M, T, B, BP),
        out_type=(
            jax.ShapeDtypeStruct((N,), jnp.float32),
            jax.ShapeDtypeStruct((B,), jnp.float32),
        ),
        mesh=mesh,
        compiler_params=pltpu.CompilerParams(needs_layout_passes=False),
        scratch_types=[
            pltpu.VMEM((T,), jnp.int32),
            pltpu.VMEM((T,), jnp.int32),
            pltpu.VMEM((T,), jnp.float32),
            pltpu.VMEM((T,), jnp.float32),
            pltpu.VMEM((T,), jnp.float32),
            pltpu.VMEM((T,), jnp.float32),
            pltpu.VMEM((NW * B,), jnp.float32),
            pltpu.VMEM((L * BP,), jnp.float32),
            pltpu.VMEM((B,), jnp.float32),
            pltpu.VMEM((B,), jnp.float32),
            pltpu.SemaphoreType.DMA((2,)),
            pltpu.SemaphoreType.DMA((2,)),
        ],
    )
    out, rawq = pass2(seg, qa, Q.astype(jnp.float32), sums, cnts)
    return (out, rawq)


# speedup vs baseline: 1.6314x; 1.3343x over previous
import functools

import jax
import jax.numpy as jnp
from jax import lax
from jax.experimental import pallas as pl
from jax.experimental.pallas import tpu as pltpu
from jax.experimental.pallas import tpu_sc as plsc

NC = 2
NS = 16
NW = NC * NS
L = 16
U = 5


def _wid():
    return lax.axis_index("s") * NC + lax.axis_index("c")


def _pass1_body(M, T, B, BP, seg_hbm, qa_hbm, sums_hbm, cnts_hbm,
                seg0, seg1, qa0, qa1, accs_a, accs_b, accc_a, accc_b,
                row_v, sems):
    wid = _wid()
    base = wid * M
    lane_off = lax.iota(jnp.int32, L) * BP
    ones = jnp.ones((L,), jnp.float32)
    zeros = jnp.zeros((L,), jnp.float32)
    bufs = ((seg0, qa0), (seg1, qa1))
    nchunks = M // T

    def issue(k, slot):
        off = base + k * T
        sb, qb = bufs[slot]
        c1 = pltpu.async_copy(seg_hbm.at[pl.ds(off, T)], sb, sems.at[slot])
        c2 = pltpu.async_copy(qa_hbm.at[pl.ds(off, T)], qb, sems.at[slot])
        return (c1, c2)

    copies = [issue(0, 0), None]

    @plsc.parallel_loop(0, (L * BP) // L, unroll=U)
    def zero_body(j):
        sl = pl.ds(j * L, L)
        accs_a[sl] = zeros
        accs_b[sl] = zeros
        accc_a[sl] = zeros
        accc_b[sl] = zeros

    for k in range(nchunks):
        slot = k % 2
        if k + 1 < nchunks:
            copies[(k + 1) % 2] = issue(k + 1, (k + 1) % 2)
        for c in copies[slot]:
            c.wait()
        sb, qb = bufs[slot]

        @plsc.parallel_loop(0, T // L, U, unroll=2)
        def vec_body(v0):
            for u in range(U):
                sl = pl.ds((v0 + u) * L, L)
                s = sb[sl]
                q = qb[sl]
                idx = lane_off + s
                plsc.addupdate_scatter(accs_a if u % 2 == 0 else accs_b,
                                       [idx], q)
                plsc.addupdate_scatter(accc_a if u % 2 == 0 else accc_b,
                                       [idx], ones)

    @plsc.parallel_loop(0, B // L, unroll=2)
    def red_sums(j):
        sl = pl.ds(j * L, L)
        tot = accs_a[sl] + accs_b[sl]
        for i in range(1, L):
            tot = tot + accs_a[pl.ds(i * BP + j * L, L)]
            tot = tot + accs_b[pl.ds(i * BP + j * L, L)]
        row_v[sl] = tot

    pltpu.sync_copy(row_v, sums_hbm.at[pl.ds(wid * B, B)])

    @plsc.parallel_loop(0, B // L, unroll=2)
    def red_cnts(j):
        sl = pl.ds(j * L, L)
        tot = accc_a[sl] + accc_b[sl]
        for i in range(1, L):
            tot = tot + accc_a[pl.ds(i * BP + j * L, L)]
            tot = tot + accc_b[pl.ds(i * BP + j * L, L)]
        row_v[sl] = tot

    pltpu.sync_copy(row_v, cnts_hbm.at[pl.ds(wid * B, B)])


def _pass2_body(M, T, B, BP, seg_hbm, qa_hbm, q_hbm, sums_hbm, cnts_hbm,
                out_hbm, rawq_hbm, seg0, seg1, qa0, qa1, out0, out1,
                big_v, corr_rep, qv_v, raw_v, sems, osems):
    wid = _wid()
    base = wid * M
    lane_off = lax.iota(jnp.int32, L) * BP
    bufs = ((seg0, qa0, out0), (seg1, qa1, out1))
    nchunks = M // T

    def issue(k, slot):
        off = base + k * T
        sb, qb, _ = bufs[slot]
        c1 = pltpu.async_copy(seg_hbm.at[pl.ds(off, T)], sb, sems.at[slot])
        c2 = pltpu.async_copy(qa_hbm.at[pl.ds(off, T)], qb, sems.at[slot])
        return (c1, c2)

    copies = [issue(0, 0), None]

    pltpu.sync_copy(q_hbm, qv_v)
    pltpu.sync_copy(sums_hbm, big_v)

    @plsc.parallel_loop(0, B // L, unroll=2)
    def comb_sums(j):
        sl = pl.ds(j * L, L)
        s = big_v[sl]
        for i in range(1, NW):
            s = s + big_v[pl.ds(i * B + j * L, L)]
        raw_v[sl] = s

    pltpu.sync_copy(cnts_hbm, big_v)

    @plsc.parallel_loop(0, B // L, unroll=2)
    def comb_cnts(j):
        sl = pl.ds(j * L, L)
        c = big_v[sl]
        for i in range(1, NW):
            c = c + big_v[pl.ds(i * B + j * L, L)]
        corr = (qv_v[sl] - raw_v[sl]) / c
        for i in range(L):
            corr_rep[pl.ds(i * BP + j * L, L)] = corr

    @pl.when(wid == 0)
    def _():
        pltpu.sync_copy(raw_v, rawq_hbm)

    outcp = [None, None]
    for k in range(nchunks):
        slot = k % 2
        if k + 1 < nchunks:
            copies[(k + 1) % 2] = issue(k + 1, (k + 1) % 2)
        for c in copies[slot]:
            c.wait()
        if outcp[slot] is not None:
            outcp[slot].wait()
        sb, qb, ob = bufs[slot]

        @plsc.parallel_loop(0, T // L, unroll=U)
        def vec_body(v):
            sl = pl.ds(v * L, L)
            s = sb[sl]
            q = qb[sl]
            c = plsc.load_gather(corr_rep, [lane_off + s])
            ob[sl] = q + c

        off = base + k * T
        outcp[slot] = pltpu.async_copy(ob, out_hbm.at[pl.ds(off, T)],
                                       osems.at[slot])
    for cp in outcp:
        if cp is not None:
            cp.wait()


def kernel(Za, Qa, Q, batch_seg):
    del Za
    N = Qa.shape[0]
    B = Q.shape[0]
    assert N % NW == 0
    M = N // NW
    T = 10000
    assert M % T == 0 and (T // L) % U == 0

    seg = batch_seg.astype(jnp.int32)
    qa = Qa.astype(jnp.float32)

    BP = B + 1

    mesh = plsc.VectorSubcoreMesh(core_axis_name="c", subcore_axis_name="s")

    pass1 = pl.kernel(
        functools.partial(_pass1_body, M, T, B, BP),
        out_type=(
            jax.ShapeDtypeStruct((NW * B,), jnp.float32),
            jax.ShapeDtypeStruct((NW * B,), jnp.float32),
        ),
        mesh=mesh,
        compiler_params=pltpu.CompilerParams(needs_layout_passes=False),
        scratch_types=[
            pltpu.VMEM((T,), jnp.int32),
            pltpu.VMEM((T,), jnp.int32),
            pltpu.VMEM((T,), jnp.float32),
            pltpu.VMEM((T,), jnp.float32),
            pltpu.VMEM((L * BP,), jnp.float32),
            pltpu.VMEM((L * BP,), jnp.float32),
            pltpu.VMEM((L * BP,), jnp.float32),
            pltpu.VMEM((L * BP,), jnp.float32),
            pltpu.VMEM((B,), jnp.float32),
            pltpu.SemaphoreType.DMA((2,)),
        ],
    )
    sums, cnts = pass1(seg, qa)

    pass2 = pl.kernel(
        functools.partial(_pass2_body, M, T, B, BP),
        out_type=(
            jax.ShapeDtypeStruct((N,), jnp.float32),
            jax.ShapeDtypeStruct((B,), jnp.float32),
        ),
        mesh=mesh,
        compiler_params=pltpu.CompilerParams(needs_layout_passes=False),
        scratch_types=[
            pltpu.VMEM((T,), jnp.int32),
            pltpu.VMEM((T,), jnp.int32),
            pltpu.VMEM((T,), jnp.float32),
            pltpu.VMEM((T,), jnp.float32),
            pltpu.VMEM((T,), jnp.float32),
            pltpu.VMEM((T,), jnp.float32),
            pltpu.VMEM((NW * B,), jnp.float32),
            pltpu.VMEM((L * BP,), jnp.float32),
            pltpu.VMEM((B,), jnp.float32),
            pltpu.VMEM((B,), jnp.float32),
            pltpu.SemaphoreType.DMA((2,)),
            pltpu.SemaphoreType.DMA((2,)),
        ],
    )
    out, rawq = pass2(seg, qa, Q.astype(jnp.float32), sums, cnts)
    return (out, rawq)


# speedup vs baseline: 1.7198x; 1.0542x over previous
import functools

import jax
import jax.numpy as jnp
from jax import lax
from jax.experimental import pallas as pl
from jax.experimental.pallas import tpu as pltpu
from jax.experimental.pallas import tpu_sc as plsc

NC = 2
NS = 16
NW = NC * NS
L = 16
U = 5


def _wid():
    return lax.axis_index("s") * NC + lax.axis_index("c")


def _pass1_body(M, T, B, BP, seg_hbm, qa_hbm, sums_hbm, cnts_hbm,
                seg0, seg1, qa0, qa1, accs_a, accc_a, row_v, sems):
    wid = _wid()
    base = wid * M
    lane_off = lax.iota(jnp.int32, L) * BP
    ones = jnp.ones((L,), jnp.float32)
    zeros = jnp.zeros((L,), jnp.float32)
    bufs = ((seg0, qa0), (seg1, qa1))
    nchunks = M // T

    def issue(k, slot):
        off = base + k * T
        sb, qb = bufs[slot]
        c1 = pltpu.async_copy(seg_hbm.at[pl.ds(off, T)], sb, sems.at[slot])
        c2 = pltpu.async_copy(qa_hbm.at[pl.ds(off, T)], qb, sems.at[slot])
        return (c1, c2)

    copies = [issue(0, 0), None]

    @plsc.parallel_loop(0, (L * BP) // L, unroll=U)
    def zero_body(j):
        sl = pl.ds(j * L, L)
        accs_a[sl] = zeros
        accc_a[sl] = zeros

    for k in range(nchunks):
        slot = k % 2
        if k + 1 < nchunks:
            copies[(k + 1) % 2] = issue(k + 1, (k + 1) % 2)
        for c in copies[slot]:
            c.wait()
        sb, qb = bufs[slot]

        @plsc.parallel_loop(0, T // L, U, unroll=2)
        def vec_body(v0):
            for u in range(U):
                sl = pl.ds((v0 + u) * L, L)
                s = sb[sl]
                q = qb[sl]
                idx = lane_off + s
                plsc.addupdate_scatter(accs_a, [idx], q)
                plsc.addupdate_scatter(accc_a, [idx], ones)

    @plsc.parallel_loop(0, B // L, unroll=2)
    def red_sums(j):
        sl = pl.ds(j * L, L)
        tot = accs_a[sl]
        for i in range(1, L):
            tot = tot + accs_a[pl.ds(i * BP + j * L, L)]
        row_v[sl] = tot

    pltpu.sync_copy(row_v, sums_hbm.at[pl.ds(wid * B, B)])

    @plsc.parallel_loop(0, B // L, unroll=2)
    def red_cnts(j):
        sl = pl.ds(j * L, L)
        tot = accc_a[sl]
        for i in range(1, L):
            tot = tot + accc_a[pl.ds(i * BP + j * L, L)]
        row_v[sl] = tot

    pltpu.sync_copy(row_v, cnts_hbm.at[pl.ds(wid * B, B)])


def _pass2_body(M, T, B, BP, seg_hbm, qa_hbm, q_hbm, sums_hbm, cnts_hbm,
                out_hbm, rawq_hbm, seg0, seg1, qa0, qa1, out0, out1,
                big_v, corr_rep, qv_v, raw_v, sems, osems):
    wid = _wid()
    base = wid * M
    lane_off = lax.iota(jnp.int32, L) * BP
    bufs = ((seg0, qa0, out0), (seg1, qa1, out1))
    nchunks = M // T

    def issue(k, slot):
        off = base + k * T
        sb, qb, _ = bufs[slot]
        c1 = pltpu.async_copy(seg_hbm.at[pl.ds(off, T)], sb, sems.at[slot])
        c2 = pltpu.async_copy(qa_hbm.at[pl.ds(off, T)], qb, sems.at[slot])
        return (c1, c2)

    copies = [issue(0, 0), None]

    pltpu.sync_copy(q_hbm, qv_v)
    pltpu.sync_copy(sums_hbm, big_v)

    @plsc.parallel_loop(0, B // L, unroll=2)
    def comb_sums(j):
        sl = pl.ds(j * L, L)
        s = big_v[sl]
        for i in range(1, NW):
            s = s + big_v[pl.ds(i * B + j * L, L)]
        raw_v[sl] = s

    pltpu.sync_copy(cnts_hbm, big_v)

    @plsc.parallel_loop(0, B // L, unroll=2)
    def comb_cnts(j):
        sl = pl.ds(j * L, L)
        c = big_v[sl]
        for i in range(1, NW):
            c = c + big_v[pl.ds(i * B + j * L, L)]
        corr = (qv_v[sl] - raw_v[sl]) / c
        for i in range(L):
            corr_rep[pl.ds(i * BP + j * L, L)] = corr

    @pl.when(wid == 0)
    def _():
        pltpu.sync_copy(raw_v, rawq_hbm)

    outcp = [None, None]
    for k in range(nchunks):
        slot = k % 2
        if k + 1 < nchunks:
            copies[(k + 1) % 2] = issue(k + 1, (k + 1) % 2)
        for c in copies[slot]:
            c.wait()
        if outcp[slot] is not None:
            outcp[slot].wait()
        sb, qb, ob = bufs[slot]

        @plsc.parallel_loop(0, T // L, unroll=U)
        def vec_body(v):
            sl = pl.ds(v * L, L)
            s = sb[sl]
            q = qb[sl]
            c = plsc.load_gather(corr_rep, [lane_off + s])
            ob[sl] = q + c

        off = base + k * T
        outcp[slot] = pltpu.async_copy(ob, out_hbm.at[pl.ds(off, T)],
                                       osems.at[slot])
    for cp in outcp:
        if cp is not None:
            cp.wait()


def kernel(Za, Qa, Q, batch_seg):
    del Za
    N = Qa.shape[0]
    B = Q.shape[0]
    assert N % NW == 0
    M = N // NW
    T1 = 20000
    T2 = 10000
    assert M % T1 == 0 and (T1 // L) % U == 0
    assert M % T2 == 0 and (T2 // L) % U == 0

    seg = batch_seg.astype(jnp.int32)
    qa = Qa.astype(jnp.float32)

    BP = B + 1

    mesh = plsc.VectorSubcoreMesh(core_axis_name="c", subcore_axis_name="s")

    pass1 = pl.kernel(
        functools.partial(_pass1_body, M, T1, B, BP),
        out_type=(
            jax.ShapeDtypeStruct((NW * B,), jnp.float32),
            jax.ShapeDtypeStruct((NW * B,), jnp.float32),
        ),
        mesh=mesh,
        compiler_params=pltpu.CompilerParams(needs_layout_passes=False),
        scratch_types=[
            pltpu.VMEM((T1,), jnp.int32),
            pltpu.VMEM((T1,), jnp.int32),
            pltpu.VMEM((T1,), jnp.float32),
            pltpu.VMEM((T1,), jnp.float32),
            pltpu.VMEM((L * BP,), jnp.float32),
            pltpu.VMEM((L * BP,), jnp.float32),
            pltpu.VMEM((B,), jnp.float32),
            pltpu.SemaphoreType.DMA((2,)),
        ],
    )
    sums, cnts = pass1(seg, qa)

    pass2 = pl.kernel(
        functools.partial(_pass2_body, M, T2, B, BP),
        out_type=(
            jax.ShapeDtypeStruct((N,), jnp.float32),
            jax.ShapeDtypeStruct((B,), jnp.float32),
        ),
        mesh=mesh,
        compiler_params=pltpu.CompilerParams(needs_layout_passes=False),
        scratch_types=[
            pltpu.VMEM((T2,), jnp.int32),
            pltpu.VMEM((T2,), jnp.int32),
            pltpu.VMEM((T2,), jnp.float32),
            pltpu.VMEM((T2,), jnp.float32),
            pltpu.VMEM((T2,), jnp.float32),
            pltpu.VMEM((T2,), jnp.float32),
            pltpu.VMEM((NW * B,), jnp.float32),
            pltpu.VMEM((L * BP,), jnp.float32),
            pltpu.VMEM((B,), jnp.float32),
            pltpu.VMEM((B,), jnp.float32),
            pltpu.SemaphoreType.DMA((2,)),
            pltpu.SemaphoreType.DMA((2,)),
        ],
    )
    out, rawq = pass2(seg, qa, Q.astype(jnp.float32), sums, cnts)
    return (out, rawq)


# speedup vs baseline: 1.8096x; 1.0522x over previous
import functools

import jax
import jax.numpy as jnp
from jax import lax
from jax.experimental import pallas as pl
from jax.experimental.pallas import tpu as pltpu
from jax.experimental.pallas import tpu_sc as plsc

NC = 2
NS = 16
NW = NC * NS
L = 16
U = 5


def _wid():
    return lax.axis_index("s") * NC + lax.axis_index("c")


def _pass1_body(M, T, B, BP, seg_hbm, qa_hbm, sums_hbm, cnts_hbm,
                seg0, seg1, qa0, qa1, accs_a, accc_a, row_v, sems):
    wid = _wid()
    base = wid * M
    lane_off = lax.iota(jnp.int32, L) * BP
    ones = jnp.ones((L,), jnp.float32)
    zeros = jnp.zeros((L,), jnp.float32)
    bufs = ((seg0, qa0), (seg1, qa1))
    nchunks = M // T

    def issue(k, slot):
        off = base + k * T
        sb, qb = bufs[slot]
        c1 = pltpu.async_copy(seg_hbm.at[pl.ds(off, T)], sb, sems.at[slot])
        c2 = pltpu.async_copy(qa_hbm.at[pl.ds(off, T)], qb, sems.at[slot])
        return (c1, c2)

    copies = [issue(0, 0), None]

    @plsc.parallel_loop(0, (L * BP) // L, unroll=U)
    def zero_body(j):
        sl = pl.ds(j * L, L)
        accs_a[sl] = zeros
        accc_a[sl] = zeros

    for k in range(nchunks):
        slot = k % 2
        if k + 1 < nchunks:
            copies[(k + 1) % 2] = issue(k + 1, (k + 1) % 2)
        for c in copies[slot]:
            c.wait()
        sb, qb = bufs[slot]

        @plsc.parallel_loop(0, T // L, U, unroll=2)
        def vec_body(v0):
            for u in range(U):
                sl = pl.ds((v0 + u) * L, L)
                s = sb[sl]
                q = qb[sl]
                idx = lane_off + s
                plsc.addupdate_scatter(accs_a, [idx], q)
                plsc.addupdate_scatter(accc_a, [idx], ones)

    @plsc.parallel_loop(0, B // L, unroll=2)
    def red_sums(j):
        sl = pl.ds(j * L, L)
        tot = accs_a[sl]
        for i in range(1, L):
            tot = tot + accs_a[pl.ds(i * BP + j * L, L)]
        row_v[sl] = tot

    pltpu.sync_copy(row_v, sums_hbm.at[pl.ds(wid * B, B)])

    @plsc.parallel_loop(0, B // L, unroll=2)
    def red_cnts(j):
        sl = pl.ds(j * L, L)
        tot = accc_a[sl]
        for i in range(1, L):
            tot = tot + accc_a[pl.ds(i * BP + j * L, L)]
        row_v[sl] = tot

    pltpu.sync_copy(row_v, cnts_hbm.at[pl.ds(wid * B, B)])


def _sload(ref, i):
    return ref[pl.ds(i, L)][0]


def _count_lt(ref, n_items, x, strict):
    def body(_, c):
        lo, hi = c
        mid = (lo + hi) // 2
        v = _sload(ref, mid)
        pred = (v < x) if strict else (v <= x)
        go = hi > lo
        lo = jnp.where(go & pred, mid + 1, lo)
        hi = jnp.where(go & jnp.logical_not(pred), mid, hi)
        return (lo, hi)

    lo, _ = lax.fori_loop(0, 11, body, (jnp.int32(0), jnp.int32(n_items)))
    return lo


def _pass2_body(M, T, B, N, qa_hbm, q_hbm, sums_hbm, cnts_hbm,
                out_hbm, rawq_hbm, qa0, qa1, out0, out1,
                big_v, corr_v, cnt_v, qv_v, raw_v, starts_v, sems, osems):
    wid = _wid()
    base = wid * M
    lanes = lax.iota(jnp.int32, L)
    bufs = ((qa0, out0), (qa1, out1))
    nchunks = M // T

    def issue(k, slot):
        off = base + k * T
        qb, _ = bufs[slot]
        return pltpu.async_copy(qa_hbm.at[pl.ds(off, T)], qb, sems.at[slot])

    copies = [issue(0, 0), None]

    pltpu.sync_copy(q_hbm, qv_v)
    pltpu.sync_copy(sums_hbm, big_v)

    @plsc.parallel_loop(0, B // L, unroll=2)
    def comb_sums(j):
        sl = pl.ds(j * L, L)
        s = big_v[sl]
        for i in range(1, NW):
            s = s + big_v[pl.ds(i * B + j * L, L)]
        raw_v[sl] = s

    pltpu.sync_copy(cnts_hbm, big_v)

    @plsc.parallel_loop(0, B // L, unroll=2)
    def comb_cnts(j):
        sl = pl.ds(j * L, L)
        c = big_v[sl]
        for i in range(1, NW):
            c = c + big_v[pl.ds(i * B + j * L, L)]
        cnt_v[sl] = c
        corr_v[sl] = (qv_v[sl] - raw_v[sl]) / c

    @pl.when(wid == 0)
    def _():
        pltpu.sync_copy(raw_v, rawq_hbm)

    def cum_body(j, carry):
        sl = pl.ds(j * L, L)
        c = cnt_v[sl]
        cum = plsc.cumsum(c)
        starts_v[sl] = ((cum - c) + carry).astype(jnp.int32)
        return carry + jnp.sum(c)

    lax.fori_loop(0, B // L, cum_body, jnp.float32(0.0))
    starts_v[pl.ds(B, L)] = jnp.full((L,), N, jnp.int32)

    outcp = [None, None]
    for k in range(nchunks):
        slot = k % 2
        if k + 1 < nchunks:
            copies[(k + 1) % 2] = issue(k + 1, (k + 1) % 2)
        copies[slot].wait()
        if outcp[slot] is not None:
            outcp[slot].wait()
        qb, ob = bufs[slot]
        cb = base + k * T

        s_lo = _count_lt(starts_v, B, cb, strict=False) - 1
        s_hi = _count_lt(starts_v, B, cb + T, strict=True)

        def seg_body(s, _):
            l = jnp.maximum(_sload(starts_v, s), cb) - cb
            h = jnp.minimum(_sload(starts_v, s + 1), cb + T) - cb
            vsv = jnp.broadcast_to(_sload(corr_v, s), (L,))

            @pl.when(h > l)
            def _():
                hv = l // L
                tv = (h - 1) // L
                m = (lanes >= l - hv * L) & (lanes < h - hv * L)
                sl = pl.ds(hv * L, L)
                bv = jnp.where(l == hv * L, qb[sl], ob[sl])
                ob[sl] = bv + jnp.where(m, vsv, 0.0)

                @pl.when(tv > hv)
                def _():
                    slt = pl.ds(tv * L, L)
                    mt = lanes < h - tv * L
                    ob[slt] = qb[slt] + jnp.where(mt, vsv, 0.0)

                @plsc.parallel_loop(hv + 1, tv, unroll=4)
                def full(v):
                    slv = pl.ds(v * L, L)
                    ob[slv] = qb[slv] + vsv

            return 0

        lax.fori_loop(s_lo, s_hi, seg_body, 0)
        outcp[slot] = pltpu.async_copy(ob, out_hbm.at[pl.ds(cb, T)],
                                       osems.at[slot])
    for cp in outcp:
        if cp is not None:
            cp.wait()


def kernel(Za, Qa, Q, batch_seg):
    del Za
    N = Qa.shape[0]
    B = Q.shape[0]
    assert N % NW == 0
    M = N // NW
    T1 = 20000
    T2 = 20000
    assert M % T1 == 0 and (T1 // L) % U == 0
    assert M % T2 == 0 and T2 % L == 0

    seg = batch_seg.astype(jnp.int32)
    qa = Qa.astype(jnp.float32)

    BP = B + 1

    mesh = plsc.VectorSubcoreMesh(core_axis_name="c", subcore_axis_name="s")

    pass1 = pl.kernel(
        functools.partial(_pass1_body, M, T1, B, BP),
        out_type=(
            jax.ShapeDtypeStruct((NW * B,), jnp.float32),
            jax.ShapeDtypeStruct((NW * B,), jnp.float32),
        ),
        mesh=mesh,
        compiler_params=pltpu.CompilerParams(needs_layout_passes=False),
        scratch_types=[
            pltpu.VMEM((T1,), jnp.int32),
            pltpu.VMEM((T1,), jnp.int32),
            pltpu.VMEM((T1,), jnp.float32),
            pltpu.VMEM((T1,), jnp.float32),
            pltpu.VMEM((L * BP,), jnp.float32),
            pltpu.VMEM((L * BP,), jnp.float32),
            pltpu.VMEM((B,), jnp.float32),
            pltpu.SemaphoreType.DMA((2,)),
        ],
    )
    sums, cnts = pass1(seg, qa)

    pass2 = pl.kernel(
        functools.partial(_pass2_body, M, T2, B, N),
        out_type=(
            jax.ShapeDtypeStruct((N,), jnp.float32),
            jax.ShapeDtypeStruct((B,), jnp.float32),
        ),
        mesh=mesh,
        compiler_params=pltpu.CompilerParams(needs_layout_passes=False),
        scratch_types=[
            pltpu.VMEM((T2,), jnp.float32),
            pltpu.VMEM((T2,), jnp.float32),
            pltpu.VMEM((T2,), jnp.float32),
            pltpu.VMEM((T2,), jnp.float32),
            pltpu.VMEM((NW * B,), jnp.float32),
            pltpu.VMEM((B + L,), jnp.float32),
            pltpu.VMEM((B,), jnp.float32),
            pltpu.VMEM((B,), jnp.float32),
            pltpu.VMEM((B,), jnp.float32),
            pltpu.VMEM((B + L,), jnp.int32),
            pltpu.SemaphoreType.DMA((2,)),
            pltpu.SemaphoreType.DMA((2,)),
        ],
    )
    out, rawq = pass2(qa, Q.astype(jnp.float32), sums, cnts)
    return (out, rawq)
